# baseline (speedup 1.0000x reference)
import functools
import math

import numpy as np
import jax
import jax.numpy as jnp
from jax import lax
from jax.experimental import pallas as pl
from jax.experimental.pallas import tpu as pltpu
from jax.experimental.pallas import tpu_sc as plsc

_FMAPS = [64, 32, 16, 8, 4, 2]
_ARS = [[2], [2, 3], [2, 3], [2, 3], [2], [2]]
_SCALES = [0.07, 0.15, 0.33, 0.51, 0.69, 0.87, 1.05]
_NCLS = 81
_TOPK = 200
_IOU_TH = 0.45
_ROWS = 64
_NSEL = 16 * _ROWS
_CSEL = 256
_NPRI = 24528


def _priors_np():
    pri = []
    for k, f in enumerate(_FMAPS):
        s = _SCALES[k]
        sp = math.sqrt(s * _SCALES[k + 1])
        for i in range(f):
            for j in range(f):
                cx = (j + 0.5) / f
                cy = (i + 0.5) / f
                pri.append([cx, cy, s, s])
                pri.append([cx, cy, sp, sp])
                for ar in _ARS[k]:
                    r = math.sqrt(ar)
                    pri.append([cx, cy, s * r, s / r])
                    pri.append([cx, cy, s / r, s * r])
    return np.clip(np.asarray(pri, dtype=np.float32), 0.0, 1.0)


def _im2col(x):
    b, c, f, _ = x.shape
    xp = jnp.pad(x, ((0, 0), (0, 0), (1, 1), (1, 1)))
    cols = []
    for di in range(3):
        for dj in range(3):
            cols.append(lax.slice(xp, (0, 0, di, dj), (b, c, di + f, dj + f)))
    p = jnp.stack(cols, axis=2)
    p = p.transpose(0, 3, 4, 1, 2)
    return p.reshape(b * f * f, c * 9)


def _mm_body(x_ref, w_ref, b_ref, o_ref):
    o_ref[...] = (
        jnp.dot(x_ref[...], w_ref[...], preferred_element_type=jnp.float32)
        + b_ref[...]
    )


def _matmul_bias(x, w, b):
    m, k = x.shape
    n = w.shape[1]
    bm = min(m, 1024)
    grid = (m // bm,)
    return pl.pallas_call(
        _mm_body,
        grid=grid,
        in_specs=[
            pl.BlockSpec((bm, k), lambda i: (i, 0)),
            pl.BlockSpec((k, n), lambda i: (0, 0)),
            pl.BlockSpec((1, n), lambda i: (0, 0)),
        ],
        out_specs=pl.BlockSpec((bm, n), lambda i: (i, 0)),
        out_shape=jax.ShapeDtypeStruct((m, n), jnp.float32),
    )(x, w, b.reshape(1, n))


def _sm_dec_body(conf_ref, loc_ref, pri_ref, sc_ref, bx_ref):
    c = conf_ref[0]
    m = jnp.max(c, axis=-1, keepdims=True)
    e = jnp.exp(c - m)
    sc_ref[0] = e / jnp.sum(e, axis=-1, keepdims=True)

    l = loc_ref[0]
    pr = pri_ref[...]
    cxcy = pr[:, :2] + l[:, :2] * 0.1 * pr[:, 2:]
    wh = pr[:, 2:] * jnp.exp(l[:, 2:] * 0.2)
    bx_ref[0] = jnp.concatenate([cxcy - wh * 0.5, cxcy + wh * 0.5], axis=1)


def _softmax_decode(conf, loc, priors):
    b = conf.shape[0]
    chunk = 4088
    nj = _NPRI // chunk
    return pl.pallas_call(
        _sm_dec_body,
        grid=(b, nj),
        in_specs=[
            pl.BlockSpec((1, chunk, _NCLS), lambda i, j: (i, j, 0)),
            pl.BlockSpec((1, chunk, 4), lambda i, j: (i, j, 0)),
            pl.BlockSpec((chunk, 4), lambda i, j: (j, 0)),
        ],
        out_specs=[
            pl.BlockSpec((1, chunk, _NCLS), lambda i, j: (i, j, 0)),
            pl.BlockSpec((1, chunk, 4), lambda i, j: (i, j, 0)),
        ],
        out_shape=[
            jax.ShapeDtypeStruct((b, _NPRI, _NCLS), jnp.float32),
            jax.ShapeDtypeStruct((b, _NPRI, 4), jnp.float32),
        ],
    )(conf, loc, priors)


def _thresh_body(sc_ref, th_ref):
    bits = lax.bitcast_convert_type(sc_ref[0], jnp.int32)
    ncls = bits.shape[0]

    def body(_, lohi):
        lo, hi = lohi
        mid = lax.div(lo + hi, 2)
        cnt = jnp.sum((bits >= mid).astype(jnp.int32), axis=1, keepdims=True)
        ok = cnt >= _TOPK
        return jnp.where(ok, mid, lo), jnp.where(ok, hi, mid)

    lo0 = jnp.zeros((ncls, 1), jnp.int32)
    hi0 = jnp.full((ncls, 1), 0x40000000, jnp.int32)
    lo, _ = lax.fori_loop(0, 31, body, (lo0, hi0))
    th_ref[0] = lax.bitcast_convert_type(lo, jnp.float32).reshape(1, ncls)


def _thresholds(scores_t):
    b, ncls, _ = scores_t.shape
    return pl.pallas_call(
        _thresh_body,
        grid=(b,),
        in_specs=[pl.BlockSpec((1, ncls, _NPRI), lambda i: (i, 0, 0))],
        out_specs=pl.BlockSpec((1, 1, ncls), lambda i: (i, 0, 0)),
        out_shape=jax.ShapeDtypeStruct((b, 1, ncls), jnp.float32),
    )(scores_t)


def _sc_select(scores_t, thresh_b, boxes_flat, ntask):
    info = plsc.get_sparse_core_info()
    nc, ns = info.num_cores, info.num_subcores
    nw = nc * ns
    per_w = ntask // nw
    nv = _NPRI // 16
    mesh = plsc.VectorSubcoreMesh(core_axis_name="c", subcore_axis_name="s")

    @functools.partial(
        pl.kernel,
        mesh=mesh,
        compiler_params=pltpu.CompilerParams(
            needs_layout_passes=False, use_tc_tiling_on_sc=False),
        out_type=[
            jax.ShapeDtypeStruct((ntask, _NSEL), jnp.float32),
            jax.ShapeDtypeStruct((ntask, _NSEL), jnp.int32),
            jax.ShapeDtypeStruct((ntask, 4, _NSEL), jnp.float32),
        ],
        scratch_types=[
            pltpu.VMEM((_NPRI,), jnp.float32),
            pltpu.VMEM((_NSEL,), jnp.float32),
            pltpu.VMEM((_NSEL,), jnp.int32),
            pltpu.VMEM((4 * _NSEL,), jnp.int32),
            pltpu.VMEM((4, _NSEL), jnp.float32),
            pltpu.VMEM((16,), jnp.float32),
            pltpu.SemaphoreType.DMA,
        ],
    )
    def sc_kernel(st_hbm, tv_hbm, bx_hbm, sel_s_out, sel_i_out, sel_b_out,
                  sv, selsv, seliv, idx4v, rowsv, tvv, sem):
        wid = lax.axis_index("s") * nc + lax.axis_index("c")
        lane = lax.broadcasted_iota(jnp.int32, (16,), 0)

        def task(ti, _):
            t = wid * per_w + ti
            img = lax.div(t, _NCLS - 1)
            pltpu.sync_copy(st_hbm.at[t], sv)
            pltpu.sync_copy(tv_hbm.at[t], tvv)
            tvec = tvv[...]

            def init(i, _):
                selsv[pl.ds(i * 16, 16)] = jnp.full((16,), -1.0, jnp.float32)
                seliv[pl.ds(i * 16, 16)] = jnp.zeros((16,), jnp.int32)
                return 0

            lax.fori_loop(0, _NSEL // 16, init, 0)

            base = img * _NPRI
            lane_base = lane * _ROWS

            def scan(i, plcnt):
                v = sv[pl.ds(i * 16, 16)]
                m = v >= tvec
                pos = lane_base + jnp.minimum(plcnt, _ROWS - 1)
                plsc.store_scatter(selsv, [pos], v, mask=m)
                plsc.store_scatter(seliv, [pos], lane + (i * 16 + base), mask=m)
                return plcnt + jnp.where(m, 1, 0)

            lax.fori_loop(0, nv, scan, jnp.zeros((16,), jnp.int32))

            def mkidx(i, _):
                iv = seliv[pl.ds(i * 16, 16)] * 4
                for c in range(4):
                    idx4v[pl.ds(c * _NSEL + i * 16, 16)] = iv + c
                return 0

            lax.fori_loop(0, _NSEL // 16, mkidx, 0)
            per_row = _NSEL // 128
            for g in range(4 * per_row):
                cp = pltpu.async_copy(
                    bx_hbm.at[idx4v.at[pl.ds(g * 128, 128)]],
                    rowsv.at[g // per_row, pl.ds((g % per_row) * 128, 128)],
                    sem)
                cp.wait()

            pltpu.sync_copy(selsv, sel_s_out.at[t])
            pltpu.sync_copy(seliv, sel_i_out.at[t])
            pltpu.sync_copy(rowsv, sel_b_out.at[t])
            return 0

        lax.fori_loop(0, per_w, task, 0)

    return sc_kernel(scores_t, thresh_b, boxes_flat)


def _nms_body(sel_s_ref, sel_i_ref, sel_b_ref, tv_ref, out_ref):
    s = sel_s_ref[...]
    si = sel_i_ref[...]
    bx = sel_b_ref[...]
    th = tv_ref[...][:, 0:1]
    cdim = s.shape[0]

    real = s >= th
    c1 = jnp.where(real, 1, 0)
    k = 1
    while k < _NSEL:
        c1 = c1 + jnp.concatenate(
            [jnp.zeros((cdim, k), jnp.int32), c1[:, : _NSEL - k]], axis=1)
        k *= 2
    pos = jnp.where(real, c1 - 1, _NSEL + 7)
    pio_a = lax.broadcasted_iota(jnp.int32, (_CSEL, _NSEL), 0)
    vals = jnp.concatenate(
        [s[:, :, None], si.astype(jnp.float32)[:, :, None], bx], axis=2)
    comp = []
    for c in range(cdim):
        oh = jnp.where(pio_a == pos[c][None, :], 1.0, 0.0)
        comp.append(jnp.dot(oh, vals[c], precision=lax.Precision.HIGHEST,
                            preferred_element_type=jnp.float32))
    comp = jnp.stack(comp, axis=0)

    s2 = comp[:, :, 0]
    i2 = comp[:, :, 1]
    sa, sb = s2[:, :, None], s2[:, None, :]
    ia, ib = i2[:, :, None], i2[:, None, :]
    beats = (sa > sb) | ((sa == sb) & (ia < ib))
    rank = jnp.sum(jnp.where(beats, 1.0, 0.0), axis=1)

    pio_b = lax.broadcasted_iota(jnp.int32, (_CSEL, _CSEL), 0).astype(jnp.float32)
    rows = []
    for c in range(cdim):
        oh = jnp.where(pio_b == rank[c][None, :], 1.0, 0.0)
        rows.append(jnp.dot(oh, comp[c], precision=lax.Precision.HIGHEST,
                            preferred_element_type=jnp.float32))
    srt = jnp.stack(rows, axis=0)

    sc = srt[:, :_TOPK, 0]
    x1 = srt[:, :_TOPK, 2]
    y1 = srt[:, :_TOPK, 3]
    x2 = srt[:, :_TOPK, 4]
    y2 = srt[:, :_TOPK, 5]

    area = jnp.maximum(x2 - x1, 0.0) * jnp.maximum(y2 - y1, 0.0)
    ix1 = jnp.maximum(x1[:, :, None], x1[:, None, :])
    iy1 = jnp.maximum(y1[:, :, None], y1[:, None, :])
    ix2 = jnp.minimum(x2[:, :, None], x2[:, None, :])
    iy2 = jnp.minimum(y2[:, :, None], y2[:, None, :])
    iw = jnp.maximum(ix2 - ix1, 0.0)
    ih = jnp.maximum(iy2 - iy1, 0.0)
    inter = iw * ih
    iou = inter / (area[:, :, None] + area[:, None, :] - inter + 1e-9)
    ka = lax.broadcasted_iota(jnp.int32, (_TOPK, _TOPK), 0)[None]
    kb = lax.broadcasted_iota(jnp.int32, (_TOPK, _TOPK), 1)[None]
    sup_m = ((iou > _IOU_TH) & (ka < kb)).astype(jnp.float32)

    def cond(st):
        _, it, chg = st
        return jnp.logical_and(it < _TOPK, chg > 0.0)

    def body(st):
        keep, it, _ = st
        supd = jnp.max(sup_m * keep[:, :, None], axis=1)
        newkeep = 1.0 - supd
        chg = jnp.max(jnp.abs(newkeep - keep))
        return newkeep, it + 1, chg

    keep0 = sc * 0.0 + 1.0
    keep, _, _ = lax.while_loop(
        cond, body, (keep0, jnp.int32(0), jnp.float32(1.0)))

    out = jnp.concatenate(
        [(sc * keep)[:, :, None], srt[:, :_TOPK, 2:6]], axis=2)
    out_ref[...] = out


def _sort_nms(sel_s, sel_i, sel_b, tv):
    ntask = sel_s.shape[0]
    cblk = 8
    return pl.pallas_call(
        _nms_body,
        grid=(ntask // cblk,),
        in_specs=[
            pl.BlockSpec((cblk, _NSEL), lambda i: (i, 0)),
            pl.BlockSpec((cblk, _NSEL), lambda i: (i, 0)),
            pl.BlockSpec((cblk, _NSEL, 4), lambda i: (i, 0, 0)),
            pl.BlockSpec((cblk, 16), lambda i: (i, 0)),
        ],
        out_specs=pl.BlockSpec((cblk, _TOPK, 5), lambda i: (i, 0, 0)),
        out_shape=jax.ShapeDtypeStruct((ntask, _TOPK, 5), jnp.float32),
    )(sel_s, sel_i, sel_b, tv)


def kernel(feats, loc_w, loc_b, conf_w, conf_b):
    b = feats[0].shape[0]
    anchors = [2 + 2 * len(a) for a in _ARS]
    priors = jnp.asarray(_priors_np())

    locs, confs = [], []
    for k, f in enumerate(_FMAPS):
        a = anchors[k]
        cin9 = feats[k].shape[1] * 9
        patches = _im2col(feats[k])
        wc = jnp.concatenate(
            [loc_w[k].reshape(a * 4, cin9), conf_w[k].reshape(a * _NCLS, cin9)],
            axis=0).T
        bc = jnp.concatenate([loc_b[k], conf_b[k]])
        raw = _matmul_bias(patches, wc, bc).reshape(b, f * f, a * (4 + _NCLS))
        locs.append(raw[:, :, :a * 4].reshape(b, f * f * a, 4))
        confs.append(raw[:, :, a * 4:].reshape(b, f * f * a, _NCLS))
    loc = jnp.concatenate(locs, axis=1)
    conf = jnp.concatenate(confs, axis=1)

    scores, boxes = _softmax_decode(conf, loc, priors)

    ncls = _NCLS - 1
    scores_t = jnp.transpose(scores[:, :, 1:], (0, 2, 1))
    thresh = _thresholds(scores_t)

    ntask = b * ncls
    st_flat = scores_t.reshape(ntask, _NPRI)
    tv = jnp.broadcast_to(thresh.reshape(ntask, 1), (ntask, 16))
    sel_s, sel_i, sel_bt = _sc_select(
        st_flat, tv, boxes.reshape(b * _NPRI * 4), ntask)
    sel_b = jnp.transpose(sel_bt, (0, 2, 1))

    det = _sort_nms(sel_s, sel_i, sel_b, tv)
    return det.reshape(b, ncls, _TOPK, 5)


# speedup vs baseline: 1.0011x; 1.0011x over previous
import functools
import math

import numpy as np
import jax
import jax.numpy as jnp
from jax import lax
from jax.experimental import pallas as pl
from jax.experimental.pallas import tpu as pltpu
from jax.experimental.pallas import tpu_sc as plsc

_FMAPS = [64, 32, 16, 8, 4, 2]
_ARS = [[2], [2, 3], [2, 3], [2, 3], [2], [2]]
_SCALES = [0.07, 0.15, 0.33, 0.51, 0.69, 0.87, 1.05]
_NCLS = 81
_TOPK = 200
_IOU_TH = 0.45
_ROWS = 64
_NSEL = 16 * _ROWS
_CSEL = 256
_NPRI = 24528


def _priors_np():
    pri = []
    for k, f in enumerate(_FMAPS):
        s = _SCALES[k]
        sp = math.sqrt(s * _SCALES[k + 1])
        for i in range(f):
            for j in range(f):
                cx = (j + 0.5) / f
                cy = (i + 0.5) / f
                pri.append([cx, cy, s, s])
                pri.append([cx, cy, sp, sp])
                for ar in _ARS[k]:
                    r = math.sqrt(ar)
                    pri.append([cx, cy, s * r, s / r])
                    pri.append([cx, cy, s / r, s * r])
    return np.clip(np.asarray(pri, dtype=np.float32), 0.0, 1.0)


def _im2col(x):
    b, c, f, _ = x.shape
    xp = jnp.pad(x, ((0, 0), (0, 0), (1, 1), (1, 1)))
    cols = []
    for di in range(3):
        for dj in range(3):
            cols.append(lax.slice(xp, (0, 0, di, dj), (b, c, di + f, dj + f)))
    p = jnp.stack(cols, axis=2)
    p = p.transpose(0, 3, 4, 1, 2)
    return p.reshape(b * f * f, c * 9)


def _mm_body(x_ref, w_ref, b_ref, o_ref):
    o_ref[...] = (
        jnp.dot(x_ref[...], w_ref[...], preferred_element_type=jnp.float32)
        + b_ref[...]
    )


def _matmul_bias(x, w, b):
    m, k = x.shape
    n = w.shape[1]
    bm = min(m, 1024)
    grid = (m // bm,)
    return pl.pallas_call(
        _mm_body,
        grid=grid,
        in_specs=[
            pl.BlockSpec((bm, k), lambda i: (i, 0)),
            pl.BlockSpec((k, n), lambda i: (0, 0)),
            pl.BlockSpec((1, n), lambda i: (0, 0)),
        ],
        out_specs=pl.BlockSpec((bm, n), lambda i: (i, 0)),
        out_shape=jax.ShapeDtypeStruct((m, n), jnp.float32),
    )(x, w, b.reshape(1, n))


def _sm_dec_body(conf_ref, loc_ref, pri_ref, sc_ref, bx_ref):
    c = conf_ref[0]
    m = jnp.max(c, axis=-1, keepdims=True)
    e = jnp.exp(c - m)
    sc_ref[0] = e / jnp.sum(e, axis=-1, keepdims=True)

    l = loc_ref[0]
    pr = pri_ref[...]
    cxcy = pr[:, :2] + l[:, :2] * 0.1 * pr[:, 2:]
    wh = pr[:, 2:] * jnp.exp(l[:, 2:] * 0.2)
    bx_ref[0] = jnp.concatenate([cxcy - wh * 0.5, cxcy + wh * 0.5], axis=1)


def _softmax_decode(conf, loc, priors):
    b = conf.shape[0]
    chunk = 4088
    nj = _NPRI // chunk
    return pl.pallas_call(
        _sm_dec_body,
        grid=(b, nj),
        in_specs=[
            pl.BlockSpec((1, chunk, _NCLS), lambda i, j: (i, j, 0)),
            pl.BlockSpec((1, chunk, 4), lambda i, j: (i, j, 0)),
            pl.BlockSpec((chunk, 4), lambda i, j: (j, 0)),
        ],
        out_specs=[
            pl.BlockSpec((1, chunk, _NCLS), lambda i, j: (i, j, 0)),
            pl.BlockSpec((1, chunk, 4), lambda i, j: (i, j, 0)),
        ],
        out_shape=[
            jax.ShapeDtypeStruct((b, _NPRI, _NCLS), jnp.float32),
            jax.ShapeDtypeStruct((b, _NPRI, 4), jnp.float32),
        ],
    )(conf, loc, priors)


def _thresh_body(sc_ref, th_ref):
    bits = lax.bitcast_convert_type(sc_ref[0], jnp.int32)
    ncls = bits.shape[0]

    def body(_, lohi):
        lo, hi = lohi
        mid = lax.div(lo + hi, 2)
        cnt = jnp.sum((bits >= mid).astype(jnp.int32), axis=1, keepdims=True)
        ok = cnt >= _TOPK
        return jnp.where(ok, mid, lo), jnp.where(ok, hi, mid)

    lo0 = jnp.zeros((ncls, 1), jnp.int32)
    hi0 = jnp.full((ncls, 1), 0x40000000, jnp.int32)
    lo, _ = lax.fori_loop(0, 31, body, (lo0, hi0))
    th_ref[0] = lax.bitcast_convert_type(lo, jnp.float32).reshape(1, ncls)


def _thresholds(scores_t):
    b, ncls, _ = scores_t.shape
    return pl.pallas_call(
        _thresh_body,
        grid=(b,),
        in_specs=[pl.BlockSpec((1, ncls, _NPRI), lambda i: (i, 0, 0))],
        out_specs=pl.BlockSpec((1, 1, ncls), lambda i: (i, 0, 0)),
        out_shape=jax.ShapeDtypeStruct((b, 1, ncls), jnp.float32),
    )(scores_t)


def _sc_select(scores_t, thresh_b, boxes_flat, ntask):
    info = plsc.get_sparse_core_info()
    nc, ns = info.num_cores, info.num_subcores
    nw = nc * ns
    per_w = ntask // nw
    nv = _NPRI // 16
    mesh = plsc.VectorSubcoreMesh(core_axis_name="c", subcore_axis_name="s")

    @functools.partial(
        pl.kernel,
        mesh=mesh,
        compiler_params=pltpu.CompilerParams(
            needs_layout_passes=False, use_tc_tiling_on_sc=False),
        out_type=[
            jax.ShapeDtypeStruct((ntask, _NSEL), jnp.float32),
            jax.ShapeDtypeStruct((ntask, _NSEL), jnp.int32),
            jax.ShapeDtypeStruct((ntask, 4, _NSEL), jnp.float32),
        ],
        scratch_types=[
            pltpu.VMEM((_NPRI,), jnp.float32),
            pltpu.VMEM((_NSEL,), jnp.float32),
            pltpu.VMEM((_NSEL,), jnp.int32),
            pltpu.VMEM((4 * _NSEL,), jnp.int32),
            pltpu.VMEM((4, _NSEL), jnp.float32),
            pltpu.VMEM((16,), jnp.float32),
            pltpu.SemaphoreType.DMA,
        ],
    )
    def sc_kernel(st_hbm, tv_hbm, bx_hbm, sel_s_out, sel_i_out, sel_b_out,
                  sv, selsv, seliv, idx4v, rowsv, tvv, sem):
        wid = lax.axis_index("s") * nc + lax.axis_index("c")
        lane = lax.broadcasted_iota(jnp.int32, (16,), 0)

        def task(ti, _):
            t = wid * per_w + ti
            img = lax.div(t, _NCLS - 1)
            pltpu.sync_copy(st_hbm.at[t], sv)
            pltpu.sync_copy(tv_hbm.at[t], tvv)
            tvec = tvv[...]

            def init(i, _):
                selsv[pl.ds(i * 16, 16)] = jnp.full((16,), -1.0, jnp.float32)
                seliv[pl.ds(i * 16, 16)] = jnp.zeros((16,), jnp.int32)
                return 0

            lax.fori_loop(0, _NSEL // 16, init, 0)

            base = img * _NPRI
            lane_base = lane * _ROWS
            unroll = 3

            def scan(i, plcnt):
                for u in range(unroll):
                    j = i * unroll + u
                    v = sv[pl.ds(j * 16, 16)]
                    m = v >= tvec
                    pos = lane_base + jnp.minimum(plcnt, _ROWS - 1)
                    plsc.store_scatter(selsv, [pos], v, mask=m)
                    plsc.store_scatter(seliv, [pos], lane + (j * 16 + base),
                                       mask=m)
                    plcnt = plcnt + jnp.where(m, 1, 0)
                return plcnt

            lax.fori_loop(0, nv // unroll, scan, jnp.zeros((16,), jnp.int32))

            def mkidx(i, _):
                iv = seliv[pl.ds(i * 16, 16)] * 4
                for c in range(4):
                    idx4v[pl.ds(c * _NSEL + i * 16, 16)] = iv + c
                return 0

            lax.fori_loop(0, _NSEL // 16, mkidx, 0)
            per_row = _NSEL // 128
            copies = []
            for g in range(4 * per_row):
                copies.append(pltpu.async_copy(
                    bx_hbm.at[idx4v.at[pl.ds(g * 128, 128)]],
                    rowsv.at[g // per_row, pl.ds((g % per_row) * 128, 128)],
                    sem))
            for cp in copies:
                cp.wait()

            pltpu.sync_copy(selsv, sel_s_out.at[t])
            pltpu.sync_copy(seliv, sel_i_out.at[t])
            pltpu.sync_copy(rowsv, sel_b_out.at[t])
            return 0

        lax.fori_loop(0, per_w, task, 0)

    return sc_kernel(scores_t, thresh_b, boxes_flat)


def _nms_body(sel_s_ref, sel_i_ref, sel_b_ref, tv_ref, out_ref):
    s = sel_s_ref[...]
    si = sel_i_ref[...]
    bx = sel_b_ref[...]
    th = tv_ref[...][:, 0:1]
    cdim = s.shape[0]

    real = s >= th
    c1 = jnp.where(real, 1, 0)
    k = 1
    while k < _NSEL:
        c1 = c1 + jnp.concatenate(
            [jnp.zeros((cdim, k), jnp.int32), c1[:, : _NSEL - k]], axis=1)
        k *= 2
    pos = jnp.where(real, c1 - 1, _NSEL + 7)
    pio_a = lax.broadcasted_iota(jnp.int32, (_CSEL, _NSEL), 0)
    vals = jnp.concatenate(
        [s[:, :, None], si.astype(jnp.float32)[:, :, None], bx], axis=2)
    comp = []
    for c in range(cdim):
        oh = jnp.where(pio_a == pos[c][None, :], 1.0, 0.0)
        comp.append(jnp.dot(oh, vals[c], precision=lax.Precision.HIGHEST,
                            preferred_element_type=jnp.float32))
    comp = jnp.stack(comp, axis=0)

    s2 = comp[:, :, 0]
    i2 = comp[:, :, 1]
    sa, sb = s2[:, :, None], s2[:, None, :]
    ia, ib = i2[:, :, None], i2[:, None, :]
    beats = (sa > sb) | ((sa == sb) & (ia < ib))
    rank = jnp.sum(jnp.where(beats, 1.0, 0.0), axis=1)

    pio_b = lax.broadcasted_iota(jnp.int32, (_CSEL, _CSEL), 0).astype(jnp.float32)
    rows = []
    for c in range(cdim):
        oh = jnp.where(pio_b == rank[c][None, :], 1.0, 0.0)
        rows.append(jnp.dot(oh, comp[c], precision=lax.Precision.HIGHEST,
                            preferred_element_type=jnp.float32))
    srt = jnp.stack(rows, axis=0)

    sc = srt[:, :_TOPK, 0]
    x1 = srt[:, :_TOPK, 2]
    y1 = srt[:, :_TOPK, 3]
    x2 = srt[:, :_TOPK, 4]
    y2 = srt[:, :_TOPK, 5]

    area = jnp.maximum(x2 - x1, 0.0) * jnp.maximum(y2 - y1, 0.0)
    ix1 = jnp.maximum(x1[:, :, None], x1[:, None, :])
    iy1 = jnp.maximum(y1[:, :, None], y1[:, None, :])
    ix2 = jnp.minimum(x2[:, :, None], x2[:, None, :])
    iy2 = jnp.minimum(y2[:, :, None], y2[:, None, :])
    iw = jnp.maximum(ix2 - ix1, 0.0)
    ih = jnp.maximum(iy2 - iy1, 0.0)
    inter = iw * ih
    iou = inter / (area[:, :, None] + area[:, None, :] - inter + 1e-9)
    ka = lax.broadcasted_iota(jnp.int32, (_TOPK, _TOPK), 0)[None]
    kb = lax.broadcasted_iota(jnp.int32, (_TOPK, _TOPK), 1)[None]
    sup_m = ((iou > _IOU_TH) & (ka < kb)).astype(jnp.float32)

    def cond(st):
        _, it, chg = st
        return jnp.logical_and(it < _TOPK, chg > 0.0)

    def body(st):
        keep, it, _ = st
        supd = jnp.max(sup_m * keep[:, :, None], axis=1)
        newkeep = 1.0 - supd
        chg = jnp.max(jnp.abs(newkeep - keep))
        return newkeep, it + 1, chg

    keep0 = sc * 0.0 + 1.0
    keep, _, _ = lax.while_loop(
        cond, body, (keep0, jnp.int32(0), jnp.float32(1.0)))

    out = jnp.concatenate(
        [(sc * keep)[:, :, None], srt[:, :_TOPK, 2:6]], axis=2)
    out_ref[...] = out


def _sort_nms(sel_s, sel_i, sel_b, tv):
    ntask = sel_s.shape[0]
    cblk = 8
    return pl.pallas_call(
        _nms_body,
        grid=(ntask // cblk,),
        in_specs=[
            pl.BlockSpec((cblk, _NSEL), lambda i: (i, 0)),
            pl.BlockSpec((cblk, _NSEL), lambda i: (i, 0)),
            pl.BlockSpec((cblk, _NSEL, 4), lambda i: (i, 0, 0)),
            pl.BlockSpec((cblk, 16), lambda i: (i, 0)),
        ],
        out_specs=pl.BlockSpec((cblk, _TOPK, 5), lambda i: (i, 0, 0)),
        out_shape=jax.ShapeDtypeStruct((ntask, _TOPK, 5), jnp.float32),
    )(sel_s, sel_i, sel_b, tv)


def kernel(feats, loc_w, loc_b, conf_w, conf_b):
    b = feats[0].shape[0]
    anchors = [2 + 2 * len(a) for a in _ARS]
    priors = jnp.asarray(_priors_np())

    locs, confs = [], []
    for k, f in enumerate(_FMAPS):
        a = anchors[k]
        cin9 = feats[k].shape[1] * 9
        patches = _im2col(feats[k])
        wc = jnp.concatenate(
            [loc_w[k].reshape(a * 4, cin9), conf_w[k].reshape(a * _NCLS, cin9)],
            axis=0).T
        bc = jnp.concatenate([loc_b[k], conf_b[k]])
        raw = _matmul_bias(patches, wc, bc).reshape(b, f * f, a * (4 + _NCLS))
        locs.append(raw[:, :, :a * 4].reshape(b, f * f * a, 4))
        confs.append(raw[:, :, a * 4:].reshape(b, f * f * a, _NCLS))
    loc = jnp.concatenate(locs, axis=1)
    conf = jnp.concatenate(confs, axis=1)

    scores, boxes = _softmax_decode(conf, loc, priors)

    ncls = _NCLS - 1
    scores_t = jnp.transpose(scores[:, :, 1:], (0, 2, 1))
    thresh = _thresholds(scores_t)

    ntask = b * ncls
    st_flat = scores_t.reshape(ntask, _NPRI)
    tv = jnp.broadcast_to(thresh.reshape(ntask, 1), (ntask, 16))
    sel_s, sel_i, sel_bt = _sc_select(
        st_flat, tv, boxes.reshape(b * _NPRI * 4), ntask)
    sel_b = jnp.transpose(sel_bt, (0, 2, 1))

    det = _sort_nms(sel_s, sel_i, sel_b, tv)
    return det.reshape(b, ncls, _TOPK, 5)
